# baseline (speedup 1.0000x reference)
import functools

import jax
import jax.numpy as jnp
from jax import lax
from jax.experimental import pallas as pl
from jax.experimental.pallas import tpu as pltpu
from jax.experimental.pallas import tpu_sc as plsc

_SIZE = 2048
_NQ = 16384
_REG = 1.0

_NC = 2
_NS = 16
_NW = _NC * _NS
_QPW = _NQ // _NW
_NB = _QPW // 16
_EPW = _QPW * 4
_GCHUNK = 128
_NG = _EPW // _GCHUNK

_CORNER_OFF = (0, 4 * _SIZE, 4, 4 * _SIZE + 4)


def _sc_body(xq_hbm, yq_hbm, table_hbm, out_hbm,
             xq_v, yq_v,
             w00_v, w01_v, w10_v, w11_v, ir_v,
             i0_v, i1_v, i2_v, i3_v,
             r0_v, r1_v, r2_v, r3_v,
             out_v, sem):
    wid = lax.axis_index("s") * _NC + lax.axis_index("c")
    base = wid * _QPW
    xmax = float(_SIZE - 2)
    idx_refs = (i0_v, i1_v, i2_v, i3_v)
    row_refs = (r0_v, r1_v, r2_v, r3_v)

    pltpu.sync_copy(xq_hbm.at[pl.ds(base, _QPW)], xq_v)
    pltpu.sync_copy(yq_hbm.at[pl.ds(base, _QPW)], yq_v)

    for b in range(_NB):
        sl = pl.ds(b * 16, 16)
        xq = xq_v[sl]
        yq = yq_v[sl]
        inr = ((xq >= 0.0) & (xq <= xmax) & (yq >= 0.0) & (yq <= xmax))
        xc = jnp.minimum(jnp.maximum(xq, 0.0), xmax)
        yc = jnp.minimum(jnp.maximum(yq, 0.0), xmax)
        xi = xc.astype(jnp.int32)
        yi = yc.astype(jnp.int32)
        xf = xc - xi.astype(jnp.float32)
        yf = yc - yi.astype(jnp.float32)
        xi = xi + jnp.where(xf > 0.5, 1, 0)
        yi = yi + jnp.where(yf > 0.5, 1, 0)
        xw = (xi + 1).astype(jnp.float32) - xc
        yw = (yi + 1).astype(jnp.float32) - yc
        oxw = 1.0 - xw
        oyw = 1.0 - yw
        w00_v[sl] = oxw * oyw
        w01_v[sl] = xw * oyw
        w10_v[sl] = oxw * yw
        w11_v[sl] = xw * yw
        ir_v[sl] = jnp.where(inr, 1.0, 0.0)
        e00 = (xi * _SIZE + yi) * 4
        for c in range(4):
            ebase = e00 + _CORNER_OFF[c]
            for e in range(4):
                idx_refs[c][pl.ds(e * _QPW + b * 16, 16)] = ebase + e

    copies = []
    for c in range(4):
        for j in range(_NG):
            csl = pl.ds(j * _GCHUNK, _GCHUNK)
            copies.append(
                pltpu.async_copy(table_hbm.at[idx_refs[c].at[csl]],
                                 row_refs[c].at[csl], sem))
    for cp in copies:
        cp.wait()

    for b in range(_NB):
        sl = pl.ds(b * 16, 16)
        w00 = w00_v[sl]
        w01 = w01_v[sl]
        w10 = w10_v[sl]
        w11 = w11_v[sl]
        inr = ir_v[sl] > 0.5
        for e in range(4):
            esl = pl.ds(e * _QPW + b * 16, 16)
            acc = w00 * r0_v[esl]
            acc = acc + w01 * r1_v[esl]
            acc = acc + w10 * r2_v[esl]
            acc = acc + w11 * r3_v[esl]
            eye = jnp.float32(_REG if e in (0, 3) else 0.0)
            out_v[esl] = jnp.where(inr, acc, eye)

    for e in range(4):
        pltpu.sync_copy(out_v.at[pl.ds(e * _QPW, _QPW)],
                        out_hbm.at[pl.ds(e * _NQ + base, _QPW)])


@jax.jit
def _sc_interp(xq, yq, table):
    mesh = plsc.VectorSubcoreMesh(core_axis_name="c", subcore_axis_name="s")
    f = functools.partial(
        pl.kernel,
        out_type=jax.ShapeDtypeStruct((4 * _NQ,), jnp.float32),
        mesh=mesh,
        scratch_types=[
            pltpu.VMEM((_QPW,), jnp.float32),
            pltpu.VMEM((_QPW,), jnp.float32),
            pltpu.VMEM((_QPW,), jnp.float32),
            pltpu.VMEM((_QPW,), jnp.float32),
            pltpu.VMEM((_QPW,), jnp.float32),
            pltpu.VMEM((_QPW,), jnp.float32),
            pltpu.VMEM((_QPW,), jnp.float32),
            pltpu.VMEM((_EPW,), jnp.int32),
            pltpu.VMEM((_EPW,), jnp.int32),
            pltpu.VMEM((_EPW,), jnp.int32),
            pltpu.VMEM((_EPW,), jnp.int32),
            pltpu.VMEM((_EPW,), jnp.float32),
            pltpu.VMEM((_EPW,), jnp.float32),
            pltpu.VMEM((_EPW,), jnp.float32),
            pltpu.VMEM((_EPW,), jnp.float32),
            pltpu.VMEM((_EPW,), jnp.float32),
            pltpu.SemaphoreType.DMA,
        ],
    )(_sc_body)
    return f(xq, yq, table)


def kernel(q, x, y, g_inv):
    qt = q.T
    xq = qt[0]
    yq = qt[1]
    table = g_inv.reshape(_SIZE * _SIZE * 4)
    out = _sc_interp(xq, yq, table)
    return out.reshape(4, _NQ).T.reshape(_NQ, 2, 2)


# speedup vs baseline: 449.9711x; 449.9711x over previous
import functools

import jax
import jax.numpy as jnp
from jax import lax
from jax.experimental import pallas as pl
from jax.experimental.pallas import tpu as pltpu
from jax.experimental.pallas import tpu_sc as plsc

_SIZE = 2048
_NQ = 16384
_REG = 1.0

_NC = 2
_NS = 16
_NW = _NC * _NS
_QPW = _NQ // _NW
_NB = _QPW // 16
_EPW = _QPW * 4
_GCHUNK = 128
_NG = _EPW // _GCHUNK

_EOFF = (0, 128, 4096, 4096 + 128)
_XSTRIDE = 4 * _SIZE


def _sc_body(xq_hbm, yq_hbm, ginv_hbm, out_hbm,
             xq_v, yq_v,
             w00_v, w01_v, w10_v, w11_v, ir_v,
             i0_v, i1_v, i2_v, i3_v,
             r0_v, r1_v, r2_v, r3_v,
             out_v, sem):
    wid = lax.axis_index("s") * _NC + lax.axis_index("c")
    base = wid * _QPW
    xmax = float(_SIZE - 2)
    table_hbm = ginv_hbm
    idx_refs = (i0_v, i1_v, i2_v, i3_v)
    row_refs = (r0_v, r1_v, r2_v, r3_v)

    pltpu.sync_copy(xq_hbm.at[pl.ds(base, _QPW)], xq_v)
    pltpu.sync_copy(yq_hbm.at[pl.ds(base, _QPW)], yq_v)

    for b in range(_NB):
        sl = pl.ds(b * 16, 16)
        xq = xq_v[sl]
        yq = yq_v[sl]
        inr = ((xq >= 0.0) & (xq <= xmax) & (yq >= 0.0) & (yq <= xmax))
        xc = jnp.minimum(jnp.maximum(xq, 0.0), xmax)
        yc = jnp.minimum(jnp.maximum(yq, 0.0), xmax)
        xi = xc.astype(jnp.int32)
        yi = yc.astype(jnp.int32)
        xf = xc - xi.astype(jnp.float32)
        yf = yc - yi.astype(jnp.float32)
        xi = xi + jnp.where(xf > 0.5, 1, 0)
        yi = yi + jnp.where(yf > 0.5, 1, 0)
        xw = (xi + 1).astype(jnp.float32) - xc
        yw = (yi + 1).astype(jnp.float32) - yc
        oxw = 1.0 - xw
        oyw = 1.0 - yw
        w00_v[sl] = oxw * oyw
        w01_v[sl] = xw * oyw
        w10_v[sl] = oxw * yw
        w11_v[sl] = xw * yw
        ir_v[sl] = jnp.where(inr, 1.0, 0.0)
        xoff = xi * _XSTRIDE
        yi1 = yi + 1
        yp0 = ((yi >> 7) << 8) + (yi & 127)
        yp1 = ((yi1 >> 7) << 8) + (yi1 & 127)
        corner_base = (xoff + yp0, xoff + _XSTRIDE + yp0,
                       xoff + yp1, xoff + _XSTRIDE + yp1)
        for c in range(4):
            for e in range(4):
                idx_refs[c][pl.ds(e * _QPW + b * 16, 16)] = (
                    corner_base[c] + _EOFF[e])

    copies = []
    for c in range(4):
        for j in range(_NG):
            csl = pl.ds(j * _GCHUNK, _GCHUNK)
            copies.append(
                pltpu.async_copy(table_hbm.at[idx_refs[c].at[csl]],
                                 row_refs[c].at[csl], sem))
    for cp in copies:
        cp.wait()

    for b in range(_NB):
        sl = pl.ds(b * 16, 16)
        w00 = w00_v[sl]
        w01 = w01_v[sl]
        w10 = w10_v[sl]
        w11 = w11_v[sl]
        inr = ir_v[sl] > 0.5
        for e in range(4):
            esl = pl.ds(e * _QPW + b * 16, 16)
            acc = w00 * r0_v[esl]
            acc = acc + w01 * r1_v[esl]
            acc = acc + w10 * r2_v[esl]
            acc = acc + w11 * r3_v[esl]
            eye = jnp.float32(_REG if e in (0, 3) else 0.0)
            out_v[esl] = jnp.where(inr, acc, eye)

    for e in range(4):
        pltpu.sync_copy(out_v.at[pl.ds(e * _QPW, _QPW)],
                        out_hbm.at[pl.ds(e * _NQ + base, _QPW)])


@jax.jit
def _sc_interp(xq, yq, ginv):
    mesh = plsc.VectorSubcoreMesh(core_axis_name="c", subcore_axis_name="s")
    f = functools.partial(
        pl.kernel,
        out_type=jax.ShapeDtypeStruct((4 * _NQ,), jnp.float32),
        mesh=mesh,
        scratch_types=[
            pltpu.VMEM((_QPW,), jnp.float32),
            pltpu.VMEM((_QPW,), jnp.float32),
            pltpu.VMEM((_QPW,), jnp.float32),
            pltpu.VMEM((_QPW,), jnp.float32),
            pltpu.VMEM((_QPW,), jnp.float32),
            pltpu.VMEM((_QPW,), jnp.float32),
            pltpu.VMEM((_QPW,), jnp.float32),
            pltpu.VMEM((_EPW,), jnp.int32),
            pltpu.VMEM((_EPW,), jnp.int32),
            pltpu.VMEM((_EPW,), jnp.int32),
            pltpu.VMEM((_EPW,), jnp.int32),
            pltpu.VMEM((_EPW,), jnp.float32),
            pltpu.VMEM((_EPW,), jnp.float32),
            pltpu.VMEM((_EPW,), jnp.float32),
            pltpu.VMEM((_EPW,), jnp.float32),
            pltpu.VMEM((_EPW,), jnp.float32),
            pltpu.SemaphoreType.DMA,
        ],
    )(_sc_body)
    return f(xq, yq, ginv)


def kernel(q, x, y, g_inv):
    qt = q.T
    xq = qt[0]
    yq = qt[1]
    t = jnp.transpose(g_inv, (0, 2, 3, 1))
    t = t.reshape(_SIZE, 2, 2, _SIZE // 128, 128)
    t = jnp.transpose(t, (0, 1, 3, 2, 4))
    table = t.reshape(_SIZE * _SIZE * 4)
    out = _sc_interp(xq, yq, table)
    return out.reshape(4, _NQ).T.reshape(_NQ, 2, 2)


# speedup vs baseline: 496.8664x; 1.1042x over previous
import functools

import jax
import jax.numpy as jnp
from jax import lax
from jax.experimental import pallas as pl
from jax.experimental.pallas import tpu as pltpu
from jax.experimental.pallas import tpu_sc as plsc

_SIZE = 2048
_NQ = 16384
_REG = 1.0

_NC = 2
_NS = 16
_NW = _NC * _NS
_QPW = _NQ // _NW
_NT = _QPW // 128
_NB = 8
_EPW = _QPW * 4

_EOFF = (0, 128, 4096, 4096 + 128)
_XSTRIDE = 4 * _SIZE


def _sc_body(qf_hbm, table_hbm, out_hbm,
             qb_v, w00_v, w01_v, w10_v, w11_v, ir_v,
             i0_v, i1_v, i2_v, i3_v,
             r0_v, r1_v, r2_v, r3_v,
             out_v, sem0, sem1, sem2, sem3):
    wid = lax.axis_index("s") * _NC + lax.axis_index("c")
    baseqt = wid * _NT
    xmax = float(_SIZE - 2)
    idx_refs = (i0_v, i1_v, i2_v, i3_v)
    row_refs = (r0_v, r1_v, r2_v, r3_v)
    sems = (sem0, sem1, sem2, sem3)

    for t in range(_NT):
        pltpu.sync_copy(qf_hbm.at[pl.ds((baseqt + t) * 256, 256)],
                        qb_v.at[pl.ds(t * 256, 256)])

    copies = []
    for t in range(_NT):
        for bb in range(_NB):
            b = t * _NB + bb
            sl = pl.ds(b * 16, 16)
            xq = qb_v[pl.ds(t * 256 + bb * 16, 16)]
            yq = qb_v[pl.ds(t * 256 + 128 + bb * 16, 16)]
            inr = ((xq >= 0.0) & (xq <= xmax) & (yq >= 0.0) & (yq <= xmax))
            xc = jnp.minimum(jnp.maximum(xq, 0.0), xmax)
            yc = jnp.minimum(jnp.maximum(yq, 0.0), xmax)
            xi = xc.astype(jnp.int32)
            yi = yc.astype(jnp.int32)
            xf = xc - xi.astype(jnp.float32)
            yf = yc - yi.astype(jnp.float32)
            xi = xi + jnp.where(xf > 0.5, 1, 0)
            yi = yi + jnp.where(yf > 0.5, 1, 0)
            xw = (xi + 1).astype(jnp.float32) - xc
            yw = (yi + 1).astype(jnp.float32) - yc
            oxw = 1.0 - xw
            oyw = 1.0 - yw
            w00_v[sl] = oxw * oyw
            w01_v[sl] = xw * oyw
            w10_v[sl] = oxw * yw
            w11_v[sl] = xw * yw
            ir_v[sl] = jnp.where(inr, 1.0, 0.0)
            xoff = xi * _XSTRIDE
            yi1 = yi + 1
            yp0 = yi + ((yi >> 7) << 7)
            yp1 = yi1 + ((yi1 >> 7) << 7)
            corner_base = (xoff + yp0, xoff + _XSTRIDE + yp0,
                           xoff + yp1, xoff + _XSTRIDE + yp1)
            for c in range(4):
                for e in range(4):
                    idx_refs[c][pl.ds(t * 512 + e * 128 + bb * 16, 16)] = (
                        corner_base[c] + _EOFF[e])
        tsl = pl.ds(t * 512, 512)
        for c in range(4):
            copies.append(
                pltpu.async_copy(table_hbm.at[idx_refs[c].at[tsl]],
                                 row_refs[c].at[tsl], sems[t]))

    for t in range(_NT):
        for c in range(4):
            copies[t * 4 + c].wait()
        for bb in range(_NB):
            b = t * _NB + bb
            sl = pl.ds(b * 16, 16)
            w00 = w00_v[sl]
            w01 = w01_v[sl]
            w10 = w10_v[sl]
            w11 = w11_v[sl]
            inr = ir_v[sl] > 0.5
            for e in range(4):
                esl = pl.ds(t * 512 + e * 128 + bb * 16, 16)
                acc = w00 * r0_v[esl]
                acc = acc + w01 * r1_v[esl]
                acc = acc + w10 * r2_v[esl]
                acc = acc + w11 * r3_v[esl]
                eye = jnp.float32(_REG if e in (0, 3) else 0.0)
                osl = pl.ds((e >> 1) * 1024 + t * 256 + (e & 1) * 128
                            + bb * 16, 16)
                out_v[osl] = jnp.where(inr, acc, eye)

    for g0 in range(2):
        for t in range(_NT):
            pltpu.sync_copy(
                out_v.at[pl.ds(g0 * 1024 + t * 256, 256)],
                out_hbm.at[pl.ds(g0 * (_NQ * 2) + (baseqt + t) * 256, 256)])


@jax.jit
def _sc_interp(qf, table):
    mesh = plsc.VectorSubcoreMesh(core_axis_name="c", subcore_axis_name="s")
    f = functools.partial(
        pl.kernel,
        out_type=jax.ShapeDtypeStruct((4 * _NQ,), jnp.float32),
        mesh=mesh,
        scratch_types=[
            pltpu.VMEM((2 * _QPW,), jnp.float32),
            pltpu.VMEM((_QPW,), jnp.float32),
            pltpu.VMEM((_QPW,), jnp.float32),
            pltpu.VMEM((_QPW,), jnp.float32),
            pltpu.VMEM((_QPW,), jnp.float32),
            pltpu.VMEM((_QPW,), jnp.float32),
            pltpu.VMEM((_EPW,), jnp.int32),
            pltpu.VMEM((_EPW,), jnp.int32),
            pltpu.VMEM((_EPW,), jnp.int32),
            pltpu.VMEM((_EPW,), jnp.int32),
            pltpu.VMEM((_EPW,), jnp.float32),
            pltpu.VMEM((_EPW,), jnp.float32),
            pltpu.VMEM((_EPW,), jnp.float32),
            pltpu.VMEM((_EPW,), jnp.float32),
            pltpu.VMEM((_EPW,), jnp.float32),
            pltpu.SemaphoreType.DMA,
            pltpu.SemaphoreType.DMA,
            pltpu.SemaphoreType.DMA,
            pltpu.SemaphoreType.DMA,
        ],
    )(_sc_body)
    return f(qf, table)


def kernel(q, x, y, g_inv):
    qf = jnp.transpose(q.reshape(_NQ // 128, 128, 2), (0, 2, 1)).reshape(
        2 * _NQ)
    t = jnp.transpose(g_inv, (0, 2, 3, 1))
    t = t.reshape(_SIZE, 2, 2, _SIZE // 128, 128)
    t = jnp.transpose(t, (0, 1, 3, 2, 4))
    table = t.reshape(_SIZE * _SIZE * 4)
    out = _sc_interp(qf, table)
    o = out.reshape(2, _NQ // 128, 2, 128)
    return jnp.transpose(o, (1, 3, 0, 2)).reshape(_NQ, 2, 2)


# speedup vs baseline: 528.0414x; 1.0627x over previous
import functools

import jax
import jax.numpy as jnp
from jax import lax
from jax.experimental import pallas as pl
from jax.experimental.pallas import tpu as pltpu
from jax.experimental.pallas import tpu_sc as plsc

_SIZE = 2048
_NQ = 16384
_REG = 1.0

_NC = 2
_NS = 16
_NW = _NC * _NS
_QPW = _NQ // _NW
_NT = _QPW // 128
_NB = 8
_EPW = _QPW * 4

_EOFF = (0, 128, 4096, 4096 + 128)
_XSTRIDE = 4 * _SIZE


def _sc_body(qf_hbm, table_hbm, out_hbm,
             qb_v, w00_v, w01_v, w10_v, w11_v, ir_v,
             i0_v, i1_v, i2_v, i3_v,
             r0_v, r1_v, r2_v, r3_v,
             out_v, sem0, sem1, sem2, sem3):
    wid = lax.axis_index("s") * _NC + lax.axis_index("c")
    baseqt = wid * _NT
    xmax = float(_SIZE - 2)
    idx_refs = (i0_v, i1_v, i2_v, i3_v)
    row_refs = (r0_v, r1_v, r2_v, r3_v)
    sems = (sem0, sem1, sem2, sem3)

    for t in range(_NT):
        pltpu.sync_copy(qf_hbm.at[pl.ds((baseqt + t) * 256, 256)],
                        qb_v.at[pl.ds(t * 256, 256)])

    copies = []
    for t in range(_NT):
        def p1_block(bb, _, t=t):
            b = t * _NB + bb
            sl = pl.ds(b * 16, 16)
            xq = qb_v[pl.ds(t * 256 + bb * 16, 16)]
            yq = qb_v[pl.ds(t * 256 + 128 + bb * 16, 16)]
            inr = ((xq >= 0.0) & (xq <= xmax) & (yq >= 0.0) & (yq <= xmax))
            xc = jnp.minimum(jnp.maximum(xq, 0.0), xmax)
            yc = jnp.minimum(jnp.maximum(yq, 0.0), xmax)
            xi = xc.astype(jnp.int32)
            yi = yc.astype(jnp.int32)
            xf = xc - xi.astype(jnp.float32)
            yf = yc - yi.astype(jnp.float32)
            xi = xi + jnp.where(xf > 0.5, 1, 0)
            yi = yi + jnp.where(yf > 0.5, 1, 0)
            xw = (xi + 1).astype(jnp.float32) - xc
            yw = (yi + 1).astype(jnp.float32) - yc
            oxw = 1.0 - xw
            oyw = 1.0 - yw
            w00_v[sl] = oxw * oyw
            w01_v[sl] = xw * oyw
            w10_v[sl] = oxw * yw
            w11_v[sl] = xw * yw
            ir_v[sl] = jnp.where(inr, 1.0, 0.0)
            xoff = xi * _XSTRIDE
            yi1 = yi + 1
            yp0 = yi + ((yi >> 7) << 7)
            yp1 = yi1 + ((yi1 >> 7) << 7)
            corner_base = (xoff + yp0, xoff + _XSTRIDE + yp0,
                           xoff + yp1, xoff + _XSTRIDE + yp1)
            for c in range(4):
                for e in range(4):
                    idx_refs[c][pl.ds(t * 512 + e * 128 + bb * 16, 16)] = (
                        corner_base[c] + _EOFF[e])
            return _

        lax.fori_loop(0, _NB, p1_block, None)
        tsl = pl.ds(t * 512, 512)
        for c in range(4):
            copies.append(
                pltpu.async_copy(table_hbm.at[idx_refs[c].at[tsl]],
                                 row_refs[c].at[tsl], sems[t]))

    for t in range(_NT):
        for c in range(4):
            copies[t * 4 + c].wait()

        def p3_block(bb, _, t=t):
            b = t * _NB + bb
            sl = pl.ds(b * 16, 16)
            w00 = w00_v[sl]
            w01 = w01_v[sl]
            w10 = w10_v[sl]
            w11 = w11_v[sl]
            inr = ir_v[sl] > 0.5
            for e in range(4):
                esl = pl.ds(t * 512 + e * 128 + bb * 16, 16)
                acc = w00 * r0_v[esl]
                acc = acc + w01 * r1_v[esl]
                acc = acc + w10 * r2_v[esl]
                acc = acc + w11 * r3_v[esl]
                eye = jnp.float32(_REG if e in (0, 3) else 0.0)
                osl = pl.ds((e >> 1) * 1024 + t * 256 + (e & 1) * 128
                            + bb * 16, 16)
                out_v[osl] = jnp.where(inr, acc, eye)
            return _

        lax.fori_loop(0, _NB, p3_block, None)

    for g0 in range(2):
        for t in range(_NT):
            pltpu.sync_copy(
                out_v.at[pl.ds(g0 * 1024 + t * 256, 256)],
                out_hbm.at[pl.ds(g0 * (_NQ * 2) + (baseqt + t) * 256, 256)])


@jax.jit
def _sc_interp(qf, table):
    mesh = plsc.VectorSubcoreMesh(core_axis_name="c", subcore_axis_name="s")
    f = functools.partial(
        pl.kernel,
        out_type=jax.ShapeDtypeStruct((4 * _NQ,), jnp.float32),
        mesh=mesh,
        scratch_types=[
            pltpu.VMEM((2 * _QPW,), jnp.float32),
            pltpu.VMEM((_QPW,), jnp.float32),
            pltpu.VMEM((_QPW,), jnp.float32),
            pltpu.VMEM((_QPW,), jnp.float32),
            pltpu.VMEM((_QPW,), jnp.float32),
            pltpu.VMEM((_QPW,), jnp.float32),
            pltpu.VMEM((_EPW,), jnp.int32),
            pltpu.VMEM((_EPW,), jnp.int32),
            pltpu.VMEM((_EPW,), jnp.int32),
            pltpu.VMEM((_EPW,), jnp.int32),
            pltpu.VMEM((_EPW,), jnp.float32),
            pltpu.VMEM((_EPW,), jnp.float32),
            pltpu.VMEM((_EPW,), jnp.float32),
            pltpu.VMEM((_EPW,), jnp.float32),
            pltpu.VMEM((_EPW,), jnp.float32),
            pltpu.SemaphoreType.DMA,
            pltpu.SemaphoreType.DMA,
            pltpu.SemaphoreType.DMA,
            pltpu.SemaphoreType.DMA,
        ],
    )(_sc_body)
    return f(qf, table)


def kernel(q, x, y, g_inv):
    qf = jnp.transpose(q.reshape(_NQ // 128, 128, 2), (0, 2, 1)).reshape(
        2 * _NQ)
    t = jnp.transpose(g_inv, (0, 2, 3, 1))
    t = t.reshape(_SIZE, 2, 2, _SIZE // 128, 128)
    t = jnp.transpose(t, (0, 1, 3, 2, 4))
    table = t.reshape(_SIZE * _SIZE * 4)
    out = _sc_interp(qf, table)
    o = out.reshape(2, _NQ // 128, 2, 128)
    return jnp.transpose(o, (1, 3, 0, 2)).reshape(_NQ, 2, 2)


# speedup vs baseline: 557.1891x; 1.0552x over previous
import functools

import jax
import jax.numpy as jnp
from jax import lax
from jax.experimental import pallas as pl
from jax.experimental.pallas import tpu as pltpu
from jax.experimental.pallas import tpu_sc as plsc

_SIZE = 2048
_NQ = 16384
_REG = 1.0

_NC = 2
_NS = 16
_NW = _NC * _NS
_QPW = _NQ // _NW
_NT = _QPW // 128
_NB = 8
_EPW = _QPW * 4

_EOFF = (0, 128, 4096, 4096 + 128)
_XSTRIDE = 4 * _SIZE


def _sc_body(qf_hbm, table_hbm, out_hbm,
             qb_v, w00_v, w01_v, w10_v, w11_v, ir_v,
             i0_v, i1_v, i2_v, i3_v,
             r0_v, r1_v, r2_v, r3_v,
             out_v, sem0, sem1, sem2, sem3):
    wid = lax.axis_index("s") * _NC + lax.axis_index("c")
    baseqt = wid * _NT
    xmax = float(_SIZE - 2)
    idx_refs = (i0_v, i1_v, i2_v, i3_v)
    row_refs = (r0_v, r1_v, r2_v, r3_v)
    sems = (sem0, sem1, sem2, sem3)

    pltpu.sync_copy(qf_hbm.at[pl.ds(baseqt * 256, 2 * _QPW)], qb_v)

    copies = []
    for t in range(_NT):
        def p1_block(bb, _, t=t):
            b = t * _NB + bb
            sl = pl.ds(b * 16, 16)
            xq = qb_v[pl.ds(t * 256 + bb * 16, 16)]
            yq = qb_v[pl.ds(t * 256 + 128 + bb * 16, 16)]
            inr = ((xq >= 0.0) & (xq <= xmax) & (yq >= 0.0) & (yq <= xmax))
            xc = jnp.minimum(jnp.maximum(xq, 0.0), xmax)
            yc = jnp.minimum(jnp.maximum(yq, 0.0), xmax)
            xi = xc.astype(jnp.int32)
            yi = yc.astype(jnp.int32)
            xf = xc - xi.astype(jnp.float32)
            yf = yc - yi.astype(jnp.float32)
            xi = xi + jnp.where(xf > 0.5, 1, 0)
            yi = yi + jnp.where(yf > 0.5, 1, 0)
            xw = (xi + 1).astype(jnp.float32) - xc
            yw = (yi + 1).astype(jnp.float32) - yc
            oxw = 1.0 - xw
            oyw = 1.0 - yw
            w00_v[sl] = oxw * oyw
            w01_v[sl] = xw * oyw
            w10_v[sl] = oxw * yw
            w11_v[sl] = xw * yw
            ir_v[sl] = jnp.where(inr, 1.0, 0.0)
            xoff = xi * _XSTRIDE
            yi1 = yi + 1
            yp0 = yi + ((yi >> 7) << 7)
            yp1 = yi1 + ((yi1 >> 7) << 7)
            corner_base = (xoff + yp0, xoff + _XSTRIDE + yp0,
                           xoff + yp1, xoff + _XSTRIDE + yp1)
            for c in range(4):
                for e in range(4):
                    idx_refs[c][pl.ds(t * 512 + e * 128 + bb * 16, 16)] = (
                        corner_base[c] + _EOFF[e])
            return _

        lax.fori_loop(0, _NB, p1_block, None)
        tsl = pl.ds(t * 512, 512)
        for c in range(4):
            copies.append(
                pltpu.async_copy(table_hbm.at[idx_refs[c].at[tsl]],
                                 row_refs[c].at[tsl], sems[t]))

    for t in range(_NT):
        for c in range(4):
            copies[t * 4 + c].wait()

        def p3_block(bb, _, t=t):
            b = t * _NB + bb
            sl = pl.ds(b * 16, 16)
            w00 = w00_v[sl]
            w01 = w01_v[sl]
            w10 = w10_v[sl]
            w11 = w11_v[sl]
            inr = ir_v[sl] > 0.5
            for e in range(4):
                esl = pl.ds(t * 512 + e * 128 + bb * 16, 16)
                acc = w00 * r0_v[esl]
                acc = acc + w01 * r1_v[esl]
                acc = acc + w10 * r2_v[esl]
                acc = acc + w11 * r3_v[esl]
                eye = jnp.float32(_REG if e in (0, 3) else 0.0)
                osl = pl.ds((e >> 1) * 1024 + t * 256 + (e & 1) * 128
                            + bb * 16, 16)
                out_v[osl] = jnp.where(inr, acc, eye)
            return _

        lax.fori_loop(0, _NB, p3_block, None)

    for g0 in range(2):
        pltpu.sync_copy(
            out_v.at[pl.ds(g0 * 1024, 1024)],
            out_hbm.at[pl.ds(g0 * (_NQ * 2) + baseqt * 256, 1024)])


@jax.jit
def _sc_interp(qf, table):
    mesh = plsc.VectorSubcoreMesh(core_axis_name="c", subcore_axis_name="s")
    f = functools.partial(
        pl.kernel,
        out_type=jax.ShapeDtypeStruct((4 * _NQ,), jnp.float32),
        mesh=mesh,
        scratch_types=[
            pltpu.VMEM((2 * _QPW,), jnp.float32),
            pltpu.VMEM((_QPW,), jnp.float32),
            pltpu.VMEM((_QPW,), jnp.float32),
            pltpu.VMEM((_QPW,), jnp.float32),
            pltpu.VMEM((_QPW,), jnp.float32),
            pltpu.VMEM((_QPW,), jnp.float32),
            pltpu.VMEM((_EPW,), jnp.int32),
            pltpu.VMEM((_EPW,), jnp.int32),
            pltpu.VMEM((_EPW,), jnp.int32),
            pltpu.VMEM((_EPW,), jnp.int32),
            pltpu.VMEM((_EPW,), jnp.float32),
            pltpu.VMEM((_EPW,), jnp.float32),
            pltpu.VMEM((_EPW,), jnp.float32),
            pltpu.VMEM((_EPW,), jnp.float32),
            pltpu.VMEM((_EPW,), jnp.float32),
            pltpu.SemaphoreType.DMA,
            pltpu.SemaphoreType.DMA,
            pltpu.SemaphoreType.DMA,
            pltpu.SemaphoreType.DMA,
        ],
    )(_sc_body)
    return f(qf, table)


def kernel(q, x, y, g_inv):
    qf = jnp.transpose(q.reshape(_NQ // 128, 128, 2), (0, 2, 1)).reshape(
        2 * _NQ)
    t = jnp.transpose(g_inv, (0, 2, 3, 1))
    t = t.reshape(_SIZE, 2, 2, _SIZE // 128, 128)
    t = jnp.transpose(t, (0, 1, 3, 2, 4))
    table = t.reshape(_SIZE * _SIZE * 4)
    out = _sc_interp(qf, table)
    o = out.reshape(2, _NQ // 128, 2, 128)
    return jnp.transpose(o, (1, 3, 0, 2)).reshape(_NQ, 2, 2)
